# Initial kernel scaffold; baseline (speedup 1.0000x reference)
import jax
import jax.numpy as jnp
from jax.experimental import pallas as pl


def kernel(x, edge_index, W, U, c, b):
    raise NotImplementedError("write your pallas kernel here")


# speedup vs baseline: 4.5682x; 4.5682x over previous
import jax
import jax.numpy as jnp
from jax import lax
from jax.experimental import pallas as pl
from jax.experimental.pallas import tpu as pltpu
from jax.experimental.pallas import tpu_sc as plsc

N = 10000
E = 320000
F = 128
NS = 16
L = 16
EB = 128
NB = 157
E_PAD = NB * EB * NS
N_PAD = 10240
RPT = N_PAD // NS
CMC = 128
BLK = 400


def _d_body(x_ref, dum_ref, o_ref):
    o_ref[...] = jnp.dot(x_ref[...], dum_ref[...],
                         preferred_element_type=jnp.float32)


def _final_body(x_ref, b_acc_ref, s_acc_ref, cnt_ref, wd_ref, w1_ref, ws_ref,
                bias_ref, o_ref):
    acc = jnp.dot(b_acc_ref[...], wd_ref[...],
                  preferred_element_type=jnp.float32)
    acc = acc + jnp.dot(s_acc_ref[...], w1_ref[...],
                        preferred_element_type=jnp.float32)
    acc = acc + jnp.dot(x_ref[...], ws_ref[...],
                        preferred_element_type=jnp.float32)
    conv = acc / (cnt_ref[...] + 1.0) + bias_ref[...]
    o_ref[...] = x_ref[...] + jnp.maximum(conv, 0.0)


def _sc_edges(src_ref, dst_ref, x_ref, d_ref, c01_ref,
              b_out, s_out, cnt_out,
              acc, cnt_sh, rows, srcb, dstb, wb, dc, cmrg, c01v,
              sem):
    cid = lax.axis_index("c")
    wid = lax.axis_index("s")
    dloc = dc
    cntloc = dc

    def _zrow(r, _):
        for t in range(F // L):
            rows[r, pl.ds(t * L, L)] = jnp.zeros((L,), jnp.float32)
        return 0
    lax.fori_loop(0, EB, _zrow, 0)
    for k in range(RPT // EB):
        pltpu.sync_copy(rows, acc.at[pl.ds(wid * RPT + k * EB, EB)])

    @pl.when(cid == 0)
    def _():
        pltpu.sync_copy(d_ref, dloc.at[pl.ds(0, N)])

    @pl.when(cid == 1)
    def _():
        def _zcnt(r, _):
            cntloc[pl.ds(r * L, L)] = jnp.zeros((L,), jnp.float32)
            return 0
        lax.fori_loop(0, N_PAD // L, _zcnt, 0)

    pltpu.sync_copy(c01_ref, c01v)
    plsc.subcore_barrier()

    base = wid * (NB * EB)

    def _batch_common(i):
        off = base + i * EB
        pltpu.sync_copy(src_ref.at[pl.ds(off, EB)], srcb)
        pltpu.sync_copy(dst_ref.at[pl.ds(off, EB)], dstb)
        pltpu.async_copy(x_ref.at[srcb], rows, sem).wait()

    def _batch_sc0(i, _):
        _batch_common(i)
        c01 = c01v[...]
        for g in range(EB // L):
            sv = srcb[pl.ds(g * L, L)]
            dv = dstb[pl.ds(g * L, L)]
            dsv = plsc.load_gather(dloc, [sv])
            ddv = plsc.load_gather(dloc, [dv])
            z = dsv - ddv + c01
            w = 1.0 / (1.0 + jnp.exp(-z))
            w = jnp.where(sv == dv, 0.0, w)
            wb[pl.ds(g * L, L)] = w

        def _scale(g, _):
            wv = wb[pl.ds(g * L, L)]
            for j in range(L):
                wj = wv[j]
                row = g * L + j
                for t in range(F // L):
                    sl = pl.ds(t * L, L)
                    rows[row, sl] = rows[row, sl] * wj
            return 0
        lax.fori_loop(0, EB // L, _scale, 0)
        pltpu.sync_copy(rows, acc.at[dstb], add=True)
        return 0

    def _batch_sc1(i, _):
        _batch_common(i)
        ones = jnp.ones((L,), jnp.float32)
        for g in range(EB // L):
            sl = pl.ds(g * L, L)
            sv = srcb[sl]
            dv = dstb[sl]
            valid = sv != dv
            dstb[sl] = jnp.where(valid, dv, N)
            plsc.addupdate_scatter(cntloc, [dv], ones, mask=valid)
        pltpu.sync_copy(rows, acc.at[dstb], add=True)
        return 0

    @pl.when(cid == 0)
    def _():
        lax.fori_loop(0, NB, _batch_sc0, 0)

    @pl.when(cid == 1)
    def _():
        lax.fori_loop(0, NB, _batch_sc1, 0)

    @pl.when(cid == 1)
    def _():
        pltpu.sync_copy(cntloc, cnt_sh.at[wid])
    plsc.subcore_barrier()

    stripe = pl.ds(wid * RPT, RPT)

    @pl.when(cid == 0)
    def _():
        pltpu.sync_copy(acc.at[stripe], b_out.at[stripe])

    @pl.when(cid == 1)
    def _():
        pltpu.sync_copy(acc.at[stripe], s_out.at[stripe])
        for ch in range(RPT // CMC):
            cbase = wid * RPT + ch * CMC
            pltpu.sync_copy(cnt_sh.at[:, pl.ds(cbase, CMC)], cmrg)

            def _merge(k, _):
                sl = pl.ds(k * L, L)
                tot = cmrg[0, sl]
                for r in range(1, NS):
                    tot = tot + cmrg[r, sl]
                cntloc[pl.ds(ch * CMC + k * L, L)] = tot
                return 0
            lax.fori_loop(0, CMC // L, _merge, 0)
        pltpu.sync_copy(cntloc.at[pl.ds(0, RPT)], cnt_out.at[stripe])


def _run_sc(src_p, dst_p, x, d, c01):
    mesh = plsc.VectorSubcoreMesh(core_axis_name="c", subcore_axis_name="s")
    return pl.kernel(
        _sc_edges,
        out_type=(jax.ShapeDtypeStruct((N_PAD, F), jnp.float32),
                  jax.ShapeDtypeStruct((N_PAD, F), jnp.float32),
                  jax.ShapeDtypeStruct((N_PAD,), jnp.float32)),
        mesh=mesh,
        compiler_params=pltpu.CompilerParams(needs_layout_passes=False),
        scratch_types=[
            pltpu.VMEM_SHARED((N_PAD, F), jnp.float32),
            pltpu.VMEM_SHARED((NS, N_PAD), jnp.float32),
            pltpu.VMEM((EB, F), jnp.float32),
            pltpu.VMEM((EB,), jnp.int32),
            pltpu.VMEM((EB,), jnp.int32),
            pltpu.VMEM((EB,), jnp.float32),
            pltpu.VMEM((N_PAD,), jnp.float32),
            pltpu.VMEM((NS, CMC), jnp.float32),
            pltpu.VMEM((L,), jnp.float32),
            pltpu.SemaphoreType.DMA,
        ],
    )(src_p, dst_p, x, d, c01)


def kernel(x, edge_index, W, U, c, b):
    W0 = W[:F]
    W1 = W[F:]
    qs = jax.nn.softmax(c)
    wd_t = (W0 - W1).T
    w1_t = W1.T
    ws_t = (qs[0] * W0 + qs[1] * W1).T
    du = U[0] - U[1]
    dum = jnp.zeros((F, 128), jnp.float32).at[:, 0].set(du)
    c01 = jnp.full((L,), c[0] - c[1], jnp.float32)

    src_p = jnp.zeros((E_PAD,), jnp.int32).at[:E].set(edge_index[0])
    dst_p = jnp.zeros((E_PAD,), jnp.int32).at[:E].set(edge_index[1])

    dmat = pl.pallas_call(
        _d_body,
        grid=(N // BLK,),
        in_specs=[pl.BlockSpec((BLK, F), lambda i: (i, 0)),
                  pl.BlockSpec((F, 128), lambda i: (0, 0))],
        out_specs=pl.BlockSpec((BLK, 128), lambda i: (i, 0)),
        out_shape=jax.ShapeDtypeStruct((N, 128), jnp.float32),
    )(x, dum)
    d = dmat[:, 0]

    b_acc, s_acc, cnt = _run_sc(src_p, dst_p, x, d, c01)

    bb = b_acc[:N]
    ss = s_acc[:N]
    cnt_b = jnp.broadcast_to(cnt[:N, None], (N, F))

    out = pl.pallas_call(
        _final_body,
        grid=(N // BLK,),
        in_specs=[pl.BlockSpec((BLK, F), lambda i: (i, 0)),
                  pl.BlockSpec((BLK, F), lambda i: (i, 0)),
                  pl.BlockSpec((BLK, F), lambda i: (i, 0)),
                  pl.BlockSpec((BLK, F), lambda i: (i, 0)),
                  pl.BlockSpec((F, F), lambda i: (0, 0)),
                  pl.BlockSpec((F, F), lambda i: (0, 0)),
                  pl.BlockSpec((F, F), lambda i: (0, 0)),
                  pl.BlockSpec((1, F), lambda i: (0, 0))],
        out_specs=pl.BlockSpec((BLK, F), lambda i: (i, 0)),
        out_shape=jax.ShapeDtypeStruct((N, F), jnp.float32),
    )(x, bb, ss, cnt_b, wd_t, w1_t, ws_t, b.reshape(1, F))
    return out
